# baseline (speedup 1.0000x reference)
import functools

import numpy as np
import jax
import jax.numpy as jnp
from jax import lax
from jax.experimental import pallas as pl
from jax.experimental.pallas import tpu as pltpu
from jax.experimental.pallas import tpu_sc as plsc

N_LEVELS = 12
F = 8
T = 1 << 15
N_PTS = 262144
RES = [int(np.floor(16 * (1.5 ** l))) for l in range(N_LEVELS)]
P1 = np.uint32(2654435761).astype(np.int32)
P2 = np.int32(805459861)
RT = N_LEVELS * T

NC, NS = 2, 16
NW = NC * NS
PTS_W = N_PTS // NW
CH = 512
NCHUNK = PTS_W // CH
GP = CH // 16
ITERS = GP * N_LEVELS
K = 12
ED = N_LEVELS * F


def _dyng(v, idx):
    return lax.gather(
        v, idx[:, None],
        lax.GatherDimensionNumbers(offset_dims=(), collapsed_slice_dims=(0,),
                                   start_index_map=(0,)),
        slice_sizes=(1,), mode=lax.GatherScatterMode.PROMISE_IN_BOUNDS)


def _sc_encode(xT, tab2):
    mesh = plsc.VectorSubcoreMesh(core_axis_name="c", subcore_axis_name="s")

    @functools.partial(
        pl.kernel,
        out_type=jax.ShapeDtypeStruct((N_PTS * ED,), jnp.float32),
        mesh=mesh,
        compiler_params=pltpu.CompilerParams(use_tc_tiling_on_sc=False),
        scratch_types=[
            pltpu.VMEM((CH,), jnp.float32),
            pltpu.VMEM((CH,), jnp.float32),
            pltpu.VMEM((CH,), jnp.float32),
            pltpu.VMEM((K, 128), jnp.int32),
            pltpu.VMEM((K * 3 * 16,), jnp.float32),
            pltpu.VMEM((K * 128, 2 * F), jnp.float32),
            pltpu.VMEM((CH * ED,), jnp.float32),
            pltpu.SMEM((N_LEVELS,), jnp.float32),
            pltpu.SemaphoreType.DMA((K,)),
        ],
    )
    def body(xT_ref, tab_ref, enc_ref, x0b, x1b, x2b, idxbuf, ffbuf,
             rowsbuf, encbuf, res_smem, sems):
        xbufs = (x0b, x1b, x2b)
        wid = lax.axis_index("s") * NC + lax.axis_index("c")
        for l in range(N_LEVELS):
            res_smem[l] = np.float32(RES[l])
        iota = lax.iota(jnp.int32, 16)
        oddoff = (iota & 1) * RT
        patb = iota >> 3
        lowpat = iota & 7

        def fire(t, s):
            level = t % N_LEVELS
            group = t // N_LEVELS
            resf = res_smem[level]
            lvec = oddoff + level * T
            gb = group * 16
            xv = [xbufs[d][pl.ds(gb, 16)] for d in range(3)]
            pos = [xv[d] * resf for d in range(3)]
            ii = [p.astype(jnp.int32) for p in pos]
            for d in range(3):
                ffbuf[pl.ds((s * 3 + d) * 16, 16)] = (
                    pos[d] - ii[d].astype(jnp.float32))
            a = [ii[0], ii[0] + 1]
            b0 = ii[1] * P1
            b = [b0, b0 + P1]
            c0 = ii[2] * P2
            c = [c0, c0 + P2]
            for i in range(2):
                for j in range(2):
                    hij = a[i] ^ b[j]
                    for k2 in range(2):
                        ci = i * 4 + j * 2 + k2
                        h = (hij ^ c[k2]) & (T - 1)
                        idxbuf[s, pl.ds(ci * 16, 16)] = h + lvec
            pltpu.async_copy(tab_ref.at[idxbuf.at[s]],
                             rowsbuf.at[pl.ds(s * 128, 128)], sems.at[s])

        def process(t, s):
            level = t % N_LEVELS
            group = t // N_LEVELS
            pltpu.make_async_copy(tab_ref.at[idxbuf.at[s]],
                                  rowsbuf.at[pl.ds(s * 128, 128)],
                                  sems.at[s]).wait()
            ff = [ffbuf[pl.ds((s * 3 + d) * 16, 16)] for d in range(3)]
            accs = []
            for j in range(8):
                pat_j = patb + 2 * j
                fx = _dyng(ff[0], pat_j)
                fy = _dyng(ff[1], pat_j)
                fz = _dyng(ff[2], pat_j)
                v = []
                for ci in range(8):
                    r0 = rowsbuf[s * 128 + ci * 16 + 2 * j, :]
                    r1 = rowsbuf[s * 128 + ci * 16 + 2 * j + 1, :]
                    v.append(r0 + r1)
                u = [v[b] + fz * (v[b + 1] - v[b]) for b in (0, 2, 4, 6)]
                t0 = u[0] + fy * (u[1] - u[0])
                t1 = u[2] + fy * (u[3] - u[2])
                accs.append(t0 + fx * (t1 - t0))
            return level, group, accs

        def merge(evn, odd):
            lA, group, acc_a = evn
            _, _, acc_b = odd
            ebase = group * (16 * ED) + 8 * lA
            for j in range(8):
                a, bb = acc_a[j], acc_b[j]
                p0 = ebase + j * (2 * ED)
                encbuf[pl.ds(p0, 16)] = jnp.where(
                    iota < 8, a, _dyng(bb, lowpat))
                encbuf[pl.ds(p0 + ED, 16)] = jnp.where(
                    iota < 8, _dyng(a, iota | 8), bb)

        @pl.loop(0, NCHUNK)
        def _chunk(chunk):
            base = wid * PTS_W + chunk * CH
            for d in range(3):
                pltpu.sync_copy(xT_ref.at[pl.ds(d * N_PTS + base, CH)],
                                xbufs[d])
            for s in range(K):
                fire(s, s)

            @pl.loop(0, (ITERS - K) // K)
            def _main(i):
                tt = i * K
                for s in range(0, K, 2):
                    evn = process(tt + s, s)
                    fire(tt + s + K, s)
                    odd = process(tt + s + 1, s + 1)
                    fire(tt + s + 1 + K, s + 1)
                    merge(evn, odd)

            for s in range(0, K, 2):
                evn = process(ITERS - K + s, s)
                odd = process(ITERS - K + s + 1, s + 1)
                merge(evn, odd)
            pltpu.sync_copy(encbuf, enc_ref.at[pl.ds(base * ED, CH * ED)])

    return body(xT, tab2)


BN = 2048


def _mlp_body(enc_ref, w1_ref, b1_ref, w2_ref, b2_ref,
              p1_ref, o1_ref, p2_ref, o2_ref):
    enc = enc_ref[...]
    p1 = jnp.dot(enc.astype(jnp.bfloat16), w1_ref[...].astype(jnp.bfloat16),
                 preferred_element_type=jnp.float32) + b1_ref[...]
    p1_ref[...] = p1
    z1 = 10.0 * p1
    o1 = (jnp.maximum(z1, 0.0) + jnp.log1p(jnp.exp(-jnp.abs(z1)))) * 0.1
    o1_ref[...] = o1
    p2 = jnp.dot(o1.astype(jnp.bfloat16), w2_ref[...].astype(jnp.bfloat16),
                 preferred_element_type=jnp.float32) + b2_ref[...]
    p2_ref[...] = p2
    z2 = 10.0 * p2
    o2_ref[...] = (jnp.maximum(z2, 0.0) + jnp.log1p(jnp.exp(-jnp.abs(z2)))) * 0.1


def _mlp(enc, w1t, b1, w2t, b2):
    out = jax.ShapeDtypeStruct((N_PTS, 64), jnp.float32)
    return pl.pallas_call(
        _mlp_body,
        grid=(N_PTS // BN,),
        in_specs=[
            pl.BlockSpec((BN, ED), lambda i: (i, 0)),
            pl.BlockSpec((ED, 64), lambda i: (0, 0)),
            pl.BlockSpec((1, 64), lambda i: (0, 0)),
            pl.BlockSpec((64, 64), lambda i: (0, 0)),
            pl.BlockSpec((1, 64), lambda i: (0, 0)),
        ],
        out_specs=[pl.BlockSpec((BN, 64), lambda i: (i, 0))] * 4,
        out_shape=[out, out, out, out],
    )(enc, w1t, b1, w2t, b2)


def kernel(x, tables, W1, b1, W2, b2):
    xT = x.T.reshape(-1)
    tab_flat = tables.reshape(RT, F)
    zeros = jnp.zeros((RT, F), jnp.float32)
    tab2 = jnp.concatenate(
        [jnp.concatenate([tab_flat, zeros], axis=1),
         jnp.concatenate([zeros, tab_flat], axis=1)], axis=0)
    enc = _sc_encode(xT, tab2).reshape(N_PTS, ED)
    p1, o1, p2, o2 = _mlp(enc, W1.T, b1.reshape(1, 64), W2.T, b2.reshape(1, 64))
    return (enc, p1, o1, p2, o2)


# speedup vs baseline: 1.3509x; 1.3509x over previous
import functools

import numpy as np
import jax
import jax.numpy as jnp
from jax import lax
from jax.experimental import pallas as pl
from jax.experimental.pallas import tpu as pltpu
from jax.experimental.pallas import tpu_sc as plsc

N_LEVELS = 12
F = 8
T = 1 << 15
N_PTS = 262144
RES = [int(np.floor(16 * (1.5 ** l))) for l in range(N_LEVELS)]
P1 = np.uint32(2654435761).astype(np.int32)
P2 = np.int32(805459861)
RT = N_LEVELS * T

NC, NS = 2, 16
NW = NC * NS
PTS_W = N_PTS // NW
CH = 256
NCHUNK = PTS_W // CH
GP = CH // 16
ITERS = GP * N_LEVELS
K = 8
ED = N_LEVELS * F


def _dyng(v, idx):
    return lax.gather(
        v, idx[:, None],
        lax.GatherDimensionNumbers(offset_dims=(), collapsed_slice_dims=(0,),
                                   start_index_map=(0,)),
        slice_sizes=(1,), mode=lax.GatherScatterMode.PROMISE_IN_BOUNDS)


def _sc_encode(xT, tab2):
    mesh = plsc.VectorSubcoreMesh(core_axis_name="c", subcore_axis_name="s")

    @functools.partial(
        pl.kernel,
        out_type=jax.ShapeDtypeStruct((N_PTS * ED,), jnp.float32),
        mesh=mesh,
        compiler_params=pltpu.CompilerParams(use_tc_tiling_on_sc=False),
        scratch_types=[
            pltpu.VMEM((CH,), jnp.float32),
            pltpu.VMEM((CH,), jnp.float32),
            pltpu.VMEM((CH,), jnp.float32),
            pltpu.VMEM((K, 128), jnp.int32),
            pltpu.VMEM((K * 3 * 16,), jnp.float32),
            pltpu.VMEM((K * 128, 2 * F), jnp.float32),
            pltpu.VMEM((CH * ED,), jnp.float32),
            pltpu.SMEM((N_LEVELS,), jnp.float32),
            pltpu.SemaphoreType.DMA((K,)),
        ],
    )
    def body(xT_ref, tab_ref, enc_ref, x0b, x1b, x2b, idxbuf, ffbuf,
             rowsbuf, encbuf, res_smem, sems):
        xbufs = (x0b, x1b, x2b)
        wid = lax.axis_index("s") * NC + lax.axis_index("c")
        for l in range(N_LEVELS):
            res_smem[l] = np.float32(RES[l])
        iota = lax.iota(jnp.int32, 16)
        oddoff = (iota & 1) * RT
        patb = iota >> 3
        lowpat = iota & 7

        def fire(t, s):
            level = t % N_LEVELS
            group = t // N_LEVELS
            resf = res_smem[level]
            lvec = oddoff + level * T
            gb = group * 16
            xv = [xbufs[d][pl.ds(gb, 16)] for d in range(3)]
            pos = [xv[d] * resf for d in range(3)]
            ii = [p.astype(jnp.int32) for p in pos]
            for d in range(3):
                ffbuf[pl.ds((s * 3 + d) * 16, 16)] = (
                    pos[d] - ii[d].astype(jnp.float32))
            a = [ii[0], ii[0] + 1]
            b0 = ii[1] * P1
            b = [b0, b0 + P1]
            c0 = ii[2] * P2
            c = [c0, c0 + P2]
            for i in range(2):
                for j in range(2):
                    hij = a[i] ^ b[j]
                    for k2 in range(2):
                        ci = i * 4 + j * 2 + k2
                        h = (hij ^ c[k2]) & (T - 1)
                        idxbuf[s, pl.ds(ci * 16, 16)] = h + lvec
            pltpu.async_copy(tab_ref.at[idxbuf.at[s]],
                             rowsbuf.at[pl.ds(s * 128, 128)], sems.at[s])

        def process(t, s):
            level = t % N_LEVELS
            group = t // N_LEVELS
            pltpu.make_async_copy(tab_ref.at[idxbuf.at[s]],
                                  rowsbuf.at[pl.ds(s * 128, 128)],
                                  sems.at[s]).wait()
            ff = [ffbuf[pl.ds((s * 3 + d) * 16, 16)] for d in range(3)]
            accs = []
            for j in range(8):
                pat_j = patb + 2 * j
                fx = _dyng(ff[0], pat_j)
                fy = _dyng(ff[1], pat_j)
                fz = _dyng(ff[2], pat_j)
                v = []
                for ci in range(8):
                    r0 = rowsbuf[s * 128 + ci * 16 + 2 * j, :]
                    r1 = rowsbuf[s * 128 + ci * 16 + 2 * j + 1, :]
                    v.append(r0 + r1)
                u = [v[b] + fz * (v[b + 1] - v[b]) for b in (0, 2, 4, 6)]
                t0 = u[0] + fy * (u[1] - u[0])
                t1 = u[2] + fy * (u[3] - u[2])
                accs.append(t0 + fx * (t1 - t0))
            return level, group, accs

        def merge(evn, odd):
            lA, group, acc_a = evn
            _, _, acc_b = odd
            ebase = group * (16 * ED) + 8 * lA
            for j in range(8):
                a, bb = acc_a[j], acc_b[j]
                p0 = ebase + j * (2 * ED)
                encbuf[pl.ds(p0, 16)] = jnp.where(
                    iota < 8, a, _dyng(bb, lowpat))
                encbuf[pl.ds(p0 + ED, 16)] = jnp.where(
                    iota < 8, _dyng(a, iota | 8), bb)

        @pl.loop(0, NCHUNK)
        def _chunk(chunk):
            base = wid * PTS_W + chunk * CH
            for d in range(3):
                pltpu.sync_copy(xT_ref.at[pl.ds(d * N_PTS + base, CH)],
                                xbufs[d])
            for s in range(K):
                fire(s, s)

            @pl.loop(0, (ITERS - K) // K)
            def _main(i):
                tt = i * K
                for s in range(0, K, 2):
                    evn = process(tt + s, s)
                    fire(tt + s + K, s)
                    odd = process(tt + s + 1, s + 1)
                    fire(tt + s + 1 + K, s + 1)
                    merge(evn, odd)

            for s in range(0, K, 2):
                evn = process(ITERS - K + s, s)
                odd = process(ITERS - K + s + 1, s + 1)
                merge(evn, odd)
            pltpu.sync_copy(encbuf, enc_ref.at[pl.ds(base * ED, CH * ED)])

    return body(xT, tab2)


BN = 4096


def _mlp_body(enc_ref, w1_ref, b1_ref, w2_ref, b2_ref,
              p1_ref, o1_ref, p2_ref, o2_ref):
    enc = enc_ref[...]
    p1 = jnp.dot(enc.astype(jnp.bfloat16), w1_ref[...].astype(jnp.bfloat16),
                 preferred_element_type=jnp.float32) + b1_ref[...]
    p1_ref[...] = p1
    z1 = 10.0 * p1
    o1 = (jnp.maximum(z1, 0.0) + jnp.log1p(jnp.exp(-jnp.abs(z1)))) * 0.1
    o1_ref[...] = o1
    p2 = jnp.dot(o1.astype(jnp.bfloat16), w2_ref[...].astype(jnp.bfloat16),
                 preferred_element_type=jnp.float32) + b2_ref[...]
    p2_ref[...] = p2
    z2 = 10.0 * p2
    o2_ref[...] = (jnp.maximum(z2, 0.0) + jnp.log1p(jnp.exp(-jnp.abs(z2)))) * 0.1


def _mlp(enc, w1t, b1, w2t, b2):
    out = jax.ShapeDtypeStruct((N_PTS, 64), jnp.float32)
    return pl.pallas_call(
        _mlp_body,
        grid=(N_PTS // BN,),
        in_specs=[
            pl.BlockSpec((BN, ED), lambda i: (i, 0)),
            pl.BlockSpec((ED, 64), lambda i: (0, 0)),
            pl.BlockSpec((1, 64), lambda i: (0, 0)),
            pl.BlockSpec((64, 64), lambda i: (0, 0)),
            pl.BlockSpec((1, 64), lambda i: (0, 0)),
        ],
        out_specs=[pl.BlockSpec((BN, 64), lambda i: (i, 0))] * 4,
        out_shape=[out, out, out, out],
    )(enc, w1t, b1, w2t, b2)


def kernel(x, tables, W1, b1, W2, b2):
    xT = x.T.reshape(-1)
    tab_flat = tables.reshape(RT, F)
    zeros = jnp.zeros((RT, F), jnp.float32)
    tab2 = jnp.concatenate(
        [jnp.concatenate([tab_flat, zeros], axis=1),
         jnp.concatenate([zeros, tab_flat], axis=1)], axis=0)
    enc = _sc_encode(xT, tab2).reshape(N_PTS, ED)
    p1, o1, p2, o2 = _mlp(enc, W1.T, b1.reshape(1, 64), W2.T, b2.reshape(1, 64))
    return (enc, p1, o1, p2, o2)
